# baseline (speedup 1.0000x reference)
import jax
import jax.numpy as jnp
from jax.experimental import pallas as pl


def kernel(x):
    B, C, H, W = x.shape
    HW = H * W
    xf = x.reshape(B, C, HW)
    CB = 8

    def body(x_ref, o_ref):
        j = pl.program_id(1)
        o_ref[...] = x_ref[...]

        @pl.when(j == 1 // CB)
        def _():
            o_ref[0:1, 1 % CB:1 % CB + 1, :] = jnp.full((1, 1, HW), 4.0, x.dtype)

    out = pl.pallas_call(
        body,
        grid=(B, C // CB),
        in_specs=[pl.BlockSpec((1, CB, HW), lambda b, j: (b, j, 0))],
        out_specs=pl.BlockSpec((1, CB, HW), lambda b, j: (b, j, 0)),
        out_shape=jax.ShapeDtypeStruct((B, C, HW), x.dtype),
    )(xf)
    return out.reshape(B, C, H, W)


# speedup vs baseline: 3.5534x; 3.5534x over previous
import jax
import jax.numpy as jnp
from jax.experimental import pallas as pl


def kernel(x):
    B, C, H, W = x.shape
    CB = 8

    def body(x_ref, o_ref):
        j = pl.program_id(1)
        o_ref[...] = x_ref[...]

        @pl.when(j == 1 // CB)
        def _():
            o_ref[0:1, 1 % CB:1 % CB + 1, :, :] = jnp.full(
                (1, 1, H, W), 4.0, x.dtype)

    return pl.pallas_call(
        body,
        grid=(B, C // CB),
        in_specs=[pl.BlockSpec((1, CB, H, W), lambda b, j: (b, j, 0, 0))],
        out_specs=pl.BlockSpec((1, CB, H, W), lambda b, j: (b, j, 0, 0)),
        out_shape=jax.ShapeDtypeStruct((B, C, H, W), x.dtype),
    )(x)


# speedup vs baseline: 4.0077x; 1.1279x over previous
import jax
import jax.numpy as jnp
from jax.experimental import pallas as pl


def kernel(x):
    B, C, H, W = x.shape
    CB = 16

    def body(x_ref, o_ref):
        j = pl.program_id(1)
        o_ref[...] = x_ref[...]

        @pl.when(j == 1 // CB)
        def _():
            o_ref[0:1, 1 % CB:1 % CB + 1, :, :] = jnp.full(
                (1, 1, H, W), 4.0, x.dtype)

    return pl.pallas_call(
        body,
        grid=(B, C // CB),
        in_specs=[pl.BlockSpec((1, CB, H, W), lambda b, j: (b, j, 0, 0))],
        out_specs=pl.BlockSpec((1, CB, H, W), lambda b, j: (b, j, 0, 0)),
        out_shape=jax.ShapeDtypeStruct((B, C, H, W), x.dtype),
    )(x)


# speedup vs baseline: 4.0756x; 1.0169x over previous
import jax
import jax.numpy as jnp
from jax.experimental import pallas as pl


def kernel(x):
    B, C, H, W = x.shape
    CB = 32

    def body(x_ref, o_ref):
        j = pl.program_id(1)
        o_ref[...] = x_ref[...]

        @pl.when(j == 1 // CB)
        def _():
            o_ref[0:1, 1 % CB:1 % CB + 1, :, :] = jnp.full(
                (1, 1, H, W), 4.0, x.dtype)

    return pl.pallas_call(
        body,
        grid=(B, C // CB),
        in_specs=[pl.BlockSpec((1, CB, H, W), lambda b, j: (b, j, 0, 0))],
        out_specs=pl.BlockSpec((1, CB, H, W), lambda b, j: (b, j, 0, 0)),
        out_shape=jax.ShapeDtypeStruct((B, C, H, W), x.dtype),
    )(x)


# speedup vs baseline: 4.0998x; 1.0059x over previous
import jax
import jax.numpy as jnp
from jax.experimental import pallas as pl


def kernel(x):
    B, C, H, W = x.shape
    CB = 64

    def body(x_ref, o_ref):
        j = pl.program_id(1)
        o_ref[...] = x_ref[...]

        @pl.when(j == 1 // CB)
        def _():
            o_ref[0:1, 1 % CB:1 % CB + 1, :, :] = jnp.full(
                (1, 1, H, W), 4.0, x.dtype)

    return pl.pallas_call(
        body,
        grid=(B, C // CB),
        in_specs=[pl.BlockSpec((1, CB, H, W), lambda b, j: (b, j, 0, 0))],
        out_specs=pl.BlockSpec((1, CB, H, W), lambda b, j: (b, j, 0, 0)),
        out_shape=jax.ShapeDtypeStruct((B, C, H, W), x.dtype),
    )(x)
